# baseline (speedup 1.0000x reference)
import functools

import jax
import jax.numpy as jnp
from jax import lax
from jax.experimental import pallas as pl
from jax.experimental.pallas import tpu as pltpu
from jax.experimental.pallas import tpu_sc as plsc

NUM_HEADS = 32
NUM_SPATIAL = 512
B, N = 64, 128
N1 = N + 1
IP = 136
JP = 256
SPF = N * N
ABF = N1 * N1
GUARD = 16
K = 4
RSEG = 17
NSEG = (NUM_HEADS // K) * (IP // RSEG)

_info = plsc.get_sparse_core_info()
NC, NS = _info.num_cores, _info.num_subcores
NW = NC * NS
B_PER_W = B // NW


def _sc_kernel(ab_hbm, sp_hbm, tab_hbm, vd_hbm, out_hbm,
               table_v, sp_v, ab2_v, t_v, int_v, sem0, sem1):
    wid = lax.axis_index("s") * NC + lax.axis_index("c")
    pltpu.sync_copy(tab_hbm, table_v)
    pltpu.sync_copy(vd_hbm, t_v)
    iota = lax.iota(jnp.int32, 16)
    m0 = jnp.where(iota == 0, 1.0, 0.0).astype(jnp.float32)
    minv = jnp.where(iota == 0, 0.0, 1.0).astype(jnp.float32)

    def seg_body(s, carry):
        b = wid * B_PER_W + s // NSEG
        rem = lax.rem(s, NSEG)
        hblk = rem // (IP // RSEG)
        rseg = lax.rem(rem, IP // RSEG)
        parity = lax.rem(s, 2)

        @pl.when(rem == 0)
        def _setup():
            pltpu.sync_copy(sp_hbm.at[b], sp_v.at[pl.ds(GUARD, SPF)])
            pltpu.sync_copy(ab_hbm.at[b], ab2_v.at[pl.ds(0, ABF)])
            sp_v[pl.ds(0, GUARD)] = jnp.zeros((16,), jnp.int32)
            sp_v[pl.ds(GUARD + SPF, 16)] = jnp.zeros((16,), jnp.int32)

            @plsc.parallel_loop(0, (ABF + 15) // 16, unroll=4)
            def _dbl(c):
                sl = pl.ds(c * 16, 16)
                v = ab2_v[sl]
                ab2_v[sl] = v + v

        h0 = hblk * K
        i0 = rseg * RSEG
        dst = out_hbm.at[b, pl.ds(h0, K), pl.ds(i0, RSEG), :]

        @pl.when(jnp.logical_and(s >= 2, parity == 0))
        def _wait0():
            pltpu.make_async_copy(int_v.at[0], dst, sem0).wait()

        @pl.when(jnp.logical_and(s >= 2, parity == 1))
        def _wait1():
            pltpu.make_async_copy(int_v.at[1], dst, sem1).wait()

        tvals = [plsc.load_gather(t_v, [jnp.full((16,), h0 + k, jnp.int32)])
                 for k in range(K)]
        tmk = [tv * m0 for tv in tvals]

        @pl.when(rseg == 0)
        def _edge():
            for k in range(K):
                for c in range(9):
                    sl = pl.ds(c * 16, 16)
                    int_v[parity, k, 0, sl] = ab2_v[sl] + tvals[k]

        lo = jnp.maximum(i0, 1)
        hi = jnp.minimum(i0 + RSEG, N1)

        @plsc.parallel_loop(lo, hi, unroll=2)
        def _row(i):
            li = i - i0
            spb = GUARD + (i - 1) * N - 1
            abb = i * N1
            for c in range(9):
                sl16 = c * 16 if c < 8 else 120
                spv = sp_v[pl.ds(spb + sl16, 16)]
                a2 = ab2_v[pl.ds(abb + sl16, 16)]
                sp32 = spv * 32
                for k in range(K):
                    tv = plsc.load_gather(table_v, [sp32 + (h0 + k)])
                    if c == 0:
                        val = a2 + tv * minv + tmk[k]
                    else:
                        val = a2 + tv
                    int_v[parity, k, li, pl.ds(sl16, 16)] = val

        @pl.when(parity == 0)
        def _fire0():
            pltpu.make_async_copy(int_v.at[0], dst, sem0).start()

        @pl.when(parity == 1)
        def _fire1():
            pltpu.make_async_copy(int_v.at[1], dst, sem1).start()

        return carry

    lax.fori_loop(0, B_PER_W * NSEG, seg_body, 0)

    b_last = wid * B_PER_W + (B_PER_W - 1)
    dstf = out_hbm.at[b_last, pl.ds(0, K), pl.ds(0, RSEG), :]
    pltpu.make_async_copy(int_v.at[0], dstf, sem0).wait()
    pltpu.make_async_copy(int_v.at[1], dstf, sem1).wait()


def kernel(attn_bias, spatial_pos, x, spatial_table, virtual_dist):
    del x
    sp = spatial_pos.astype(jnp.int32).reshape(B, SPF)
    ab = attn_bias.reshape(B, ABF)
    tab = spatial_table.astype(jnp.float32).reshape(NUM_SPATIAL * NUM_HEADS)
    vd = virtual_dist.reshape(NUM_HEADS)

    mesh = plsc.VectorSubcoreMesh(core_axis_name="c", subcore_axis_name="s")
    run = functools.partial(
        pl.kernel,
        mesh=mesh,
        out_type=jax.ShapeDtypeStruct((B, NUM_HEADS, IP, JP), jnp.float32),
        compiler_params=pltpu.CompilerParams(
            needs_layout_passes=False, use_tc_tiling_on_sc=False),
        scratch_types=[
            pltpu.VMEM((NUM_SPATIAL * NUM_HEADS,), jnp.float32),
            pltpu.VMEM((GUARD + SPF + 16,), jnp.int32),
            pltpu.VMEM((ABF + 15,), jnp.float32),
            pltpu.VMEM((NUM_HEADS,), jnp.float32),
            pltpu.VMEM((2, K, RSEG, JP), jnp.float32),
            pltpu.SemaphoreType.DMA,
            pltpu.SemaphoreType.DMA,
        ],
    )(_sc_kernel)
    padded = run(ab, sp, tab, vd)
    return padded[:, :, :N1, :N1]


# speedup vs baseline: 1.2381x; 1.2381x over previous
import functools

import jax
import jax.numpy as jnp
from jax import lax
from jax.experimental import pallas as pl
from jax.experimental.pallas import tpu as pltpu
from jax.experimental.pallas import tpu_sc as plsc

NUM_HEADS = 32
NUM_SPATIAL = 512
B, N = 64, 128
N1 = N + 1
JP = 256
JW = 136
SPF = N * N
ABF = N1 * N1
GUARD = 16
K = 4
RSEG = 16
NSEG = (NUM_HEADS // K) * (N // RSEG)

_info = plsc.get_sparse_core_info()
NC, NS = _info.num_cores, _info.num_subcores
NW = NC * NS
B_PER_W = B // NW


def _sc_kernel(ab_hbm, sp_hbm, tab_hbm, vd_hbm, out_hbm,
               table_v, sp_v, ab2_v, t_v, r0_v, int_v, sem0, sem1, semr):
    wid = lax.axis_index("s") * NC + lax.axis_index("c")
    pltpu.sync_copy(tab_hbm, table_v)
    pltpu.sync_copy(vd_hbm, t_v)
    iota = lax.iota(jnp.int32, 16)
    m0 = jnp.where(iota == 0, 1.0, 0.0).astype(jnp.float32)
    minv = jnp.where(iota == 0, 0.0, 1.0).astype(jnp.float32)

    def seg_body(s, carry):
        b = wid * B_PER_W + s // NSEG
        rem = lax.rem(s, NSEG)
        hblk = rem // (N // RSEG)
        rseg = lax.rem(rem, N // RSEG)
        parity = lax.rem(s, 2)

        @pl.when(jnp.logical_and(s >= NSEG, rem == 0))
        def _drain_r0():
            pltpu.make_async_copy(
                r0_v.at[:, pl.ds(0, JW)],
                out_hbm.at[b, 0, :, pl.ds(0, JW)], semr).wait()

        @pl.when(rem == 0)
        def _setup():
            pltpu.sync_copy(sp_hbm.at[b], sp_v.at[pl.ds(GUARD, SPF)])
            pltpu.sync_copy(ab_hbm.at[b], ab2_v.at[pl.ds(0, ABF)])
            sp_v[pl.ds(0, GUARD)] = jnp.zeros((16,), jnp.int32)
            sp_v[pl.ds(GUARD + SPF, 16)] = jnp.zeros((16,), jnp.int32)

            @plsc.parallel_loop(0, (ABF + 15) // 16, unroll=4)
            def _dbl(c):
                sl = pl.ds(c * 16, 16)
                v = ab2_v[sl]
                ab2_v[sl] = v + v

            @plsc.parallel_loop(0, NUM_HEADS, unroll=2)
            def _edge(h):
                tval = plsc.load_gather(t_v, [jnp.full((16,), h, jnp.int32)])
                for c in range(9):
                    sl = pl.ds(c * 16, 16)
                    r0_v[h, sl] = ab2_v[sl] + tval

            pltpu.make_async_copy(
                r0_v.at[:, pl.ds(0, JW)],
                out_hbm.at[b, 0, :, pl.ds(0, JW)], semr).start()

        h0 = hblk * K
        i0 = 1 + rseg * RSEG
        dst = out_hbm.at[b, pl.ds(i0, RSEG), pl.ds(h0, K), pl.ds(0, JW)]
        src0 = int_v.at[0, :, :, pl.ds(0, JW)]
        src1 = int_v.at[1, :, :, pl.ds(0, JW)]

        @pl.when(jnp.logical_and(s >= 2, parity == 0))
        def _wait0():
            pltpu.make_async_copy(src0, dst, sem0).wait()

        @pl.when(jnp.logical_and(s >= 2, parity == 1))
        def _wait1():
            pltpu.make_async_copy(src1, dst, sem1).wait()

        tmk = [plsc.load_gather(t_v, [jnp.full((16,), h0 + k, jnp.int32)]) * m0
               for k in range(K)]

        @plsc.parallel_loop(0, RSEG, unroll=2)
        def _row(li):
            i = i0 + li
            spb = GUARD + (i - 1) * N - 1
            abb = i * N1
            for c in range(9):
                sl16 = c * 16 if c < 8 else 120
                spv = sp_v[pl.ds(spb + sl16, 16)]
                a2 = ab2_v[pl.ds(abb + sl16, 16)]
                sp32 = spv * 32
                for k in range(K):
                    tv = plsc.load_gather(table_v, [sp32 + (h0 + k)])
                    if c == 0:
                        val = a2 + tv * minv + tmk[k]
                    else:
                        val = a2 + tv
                    int_v[parity, li, k, pl.ds(sl16, 16)] = val

        @pl.when(parity == 0)
        def _fire0():
            pltpu.make_async_copy(src0, dst, sem0).start()

        @pl.when(parity == 1)
        def _fire1():
            pltpu.make_async_copy(src1, dst, sem1).start()

        return carry

    lax.fori_loop(0, B_PER_W * NSEG, seg_body, 0)

    b_last = wid * B_PER_W + (B_PER_W - 1)
    dstf = out_hbm.at[b_last, pl.ds(1, RSEG), pl.ds(0, K), pl.ds(0, JW)]
    pltpu.make_async_copy(int_v.at[0, :, :, pl.ds(0, JW)], dstf, sem0).wait()
    pltpu.make_async_copy(int_v.at[1, :, :, pl.ds(0, JW)], dstf, sem1).wait()
    pltpu.make_async_copy(r0_v.at[:, pl.ds(0, JW)],
                          out_hbm.at[b_last, 0, :, pl.ds(0, JW)], semr).wait()


def kernel(attn_bias, spatial_pos, x, spatial_table, virtual_dist):
    del x
    sp = spatial_pos.astype(jnp.int32).reshape(B, SPF)
    ab = attn_bias.reshape(B, ABF)
    tab = spatial_table.astype(jnp.float32).reshape(NUM_SPATIAL * NUM_HEADS)
    vd = virtual_dist.reshape(NUM_HEADS)

    mesh = plsc.VectorSubcoreMesh(core_axis_name="c", subcore_axis_name="s")
    run = functools.partial(
        pl.kernel,
        mesh=mesh,
        out_type=jax.ShapeDtypeStruct((B, N1, NUM_HEADS, JP), jnp.float32),
        compiler_params=pltpu.CompilerParams(
            needs_layout_passes=False, use_tc_tiling_on_sc=False),
        scratch_types=[
            pltpu.VMEM((NUM_SPATIAL * NUM_HEADS,), jnp.float32),
            pltpu.VMEM((GUARD + SPF + 16,), jnp.int32),
            pltpu.VMEM((ABF + 15,), jnp.float32),
            pltpu.VMEM((NUM_HEADS,), jnp.float32),
            pltpu.VMEM((NUM_HEADS, JW + 8), jnp.float32),
            pltpu.VMEM((2, RSEG, K, JW + 8), jnp.float32),
            pltpu.SemaphoreType.DMA,
            pltpu.SemaphoreType.DMA,
            pltpu.SemaphoreType.DMA,
        ],
    )(_sc_kernel)
    padded = run(ab, sp, tab, vd)
    return jnp.transpose(padded, (0, 2, 1, 3))[:, :, :, :N1]
